# Initial kernel scaffold; baseline (speedup 1.0000x reference)
import jax
import jax.numpy as jnp
from jax.experimental import pallas as pl


def kernel(bias_table, rel_index):
    raise NotImplementedError("write your pallas kernel here")


# speedup vs baseline: 16.7753x; 16.7753x over previous
import functools

import jax
import jax.numpy as jnp
from jax import lax
from jax.experimental import pallas as pl
from jax.experimental.pallas import tpu as pltpu
from jax.experimental.pallas import tpu_sc as plsc

NUM_HEADS = 16
N = 1024
NPOS = 3969
LANES = 16
NUM_CORES = 2
NUM_SUBCORES = 16
NUM_WORKERS = NUM_CORES * NUM_SUBCORES
ROWS_PER_TILE = N // NUM_WORKERS


@functools.partial(
    pl.kernel,
    out_type=jax.ShapeDtypeStruct((NUM_HEADS, N, N), jnp.float32),
    mesh=plsc.VectorSubcoreMesh(core_axis_name="c", subcore_axis_name="s"),
    scratch_types=[
        pltpu.VMEM((NUM_HEADS * NPOS,), jnp.float32),
        pltpu.VMEM((N,), jnp.int32),
        pltpu.VMEM((NUM_HEADS, N), jnp.float32),
    ],
    compiler_params=pltpu.CompilerParams(needs_layout_passes=False),
)
def _rel_bias_sc(table_hbm, idx_hbm, out_hbm, table_v, idx_v, out_v):
    wid = lax.axis_index("s") * NUM_CORES + lax.axis_index("c")
    pltpu.sync_copy(table_hbm, table_v)

    def row_body(c, carry):
        row = wid * ROWS_PER_TILE + c
        pltpu.sync_copy(idx_hbm.at[pl.ds(row * N, N)], idx_v)

        def vec_body(v, carry2):
            idx = idx_v[pl.ds(v * LANES, LANES)]
            for h in range(NUM_HEADS):
                vals = plsc.load_gather(table_v, [idx + h * NPOS])
                out_v[h, pl.ds(v * LANES, LANES)] = vals
            return carry2

        lax.fori_loop(0, N // LANES, vec_body, 0)
        for h in range(NUM_HEADS):
            pltpu.sync_copy(out_v.at[h], out_hbm.at[h, row])
        return carry

    lax.fori_loop(0, ROWS_PER_TILE, row_body, 0)


def kernel(bias_table, rel_index):
    table_t = jnp.transpose(bias_table).reshape(-1)
    idx = rel_index.reshape(-1)
    return _rel_bias_sc(table_t, idx)
